# baseline (speedup 1.0000x reference)
import functools

import jax
import jax.numpy as jnp
from jax.experimental import pallas as pl
from jax.experimental.pallas import tpu as pltpu

GRID_H = 16
GRID_W = 32
NODES = GRID_H * GRID_W
DIM = 64
B = 4
T = 512
ROWS = B * T
TIME_DECAY = 0.9

_HI = jax.lax.Precision.HIGHEST


def _som_kernel(z_ref, n_ref, tw_ref, som_ref, q_ref, bmu_ref,
                kl_ref, div_ref, ts_ref, nb_ref, tot_ref,
                q_scr, op_scr, rs_scr, cs_scr, ts_acc, nb_acc):
    i = pl.program_id(0)
    n = n_ref[:]
    nsq = n * n
    nn2_col = jnp.sum(nsq, axis=1, keepdims=True)
    ones_n = jnp.ones((NODES, 1), jnp.float32)
    n_aug = jnp.concatenate([n, ones_n, nn2_col], axis=1)

    @pl.when(i < B)
    def _block():
        z = z_ref[0]
        tw = tw_ref[:]
        wz = z * tw
        zn2 = jnp.sum(wz * wz, axis=1, keepdims=True)

        ones_r = jnp.ones((T, 1), jnp.float32)
        m_aug = jnp.concatenate([wz * -2.0, zn2, ones_r], axis=1)
        s = jax.lax.dot_general(
            m_aug, n_aug, (((1,), (1,)), ((), ())), precision=_HI)
        s = jnp.maximum(s, 0.0)
        dist = jnp.sqrt(s)

        one_plus = 1.0 + dist
        q_un = 1.0 / one_plus
        rs = jnp.maximum(jnp.sum(q_un, axis=1, keepdims=True), 1e-12)
        q = q_un * (1.0 / rs)
        q_ref[:] = q
        q_scr[pl.ds(i * T, T), :] = q
        op_scr[pl.ds(i * T, T), :] = one_plus
        rs_scr[pl.ds(i * T, T), :] = rs

        csum = jnp.sum(q * q, axis=0, keepdims=True)

        @pl.when(i == 0)
        def _init():
            cs_scr[:] = csum

        @pl.when(i > 0)
        def _acc():
            cs_scr[:] = cs_scr[:] + csum

        ids = jax.lax.broadcasted_iota(jnp.int32, (T, NODES), 1)
        mind = jnp.min(dist, axis=1, keepdims=True)
        bmu = jnp.min(jnp.where(dist == mind, ids, NODES), axis=1,
                      keepdims=True)
        bmu_ref[0] = jnp.transpose(bmu)

        one_hot = (ids == bmu).astype(jnp.float32).astype(jnp.bfloat16)
        gath = jax.lax.dot_general(
            one_hot, n.astype(jnp.bfloat16), (((1,), (0,)), ((), ())),
            preferred_element_type=jnp.float32)
        som_ref[0] = z + 0.1 * (gath - z)

        rid = jax.lax.broadcasted_iota(jnp.int32, (T, 1), 0)
        valid = (rid != (T - 1)).astype(jnp.float32)
        z_next = pltpu.roll(z, shift=T - 1, axis=0)
        dz = z_next - z
        ts_part = jnp.sum((dz * dz) * valid)
        bmu_next = pltpu.roll(bmu, shift=T - 1, axis=0)
        dr = jnp.abs((bmu >> 5) - (bmu_next >> 5))
        dc = jnp.abs((bmu & (GRID_W - 1)) - (bmu_next & (GRID_W - 1)))
        nb_part = jnp.sum((dr + dc).astype(jnp.float32) * valid)

        @pl.when(i == 0)
        def _init_acc():
            ts_acc[:] = jnp.reshape(ts_part, (1, 1))
            nb_acc[:] = jnp.reshape(nb_part, (1, 1))

        @pl.when(i > 0)
        def _add_acc():
            ts_acc[:] = ts_acc[:] + jnp.reshape(ts_part, (1, 1))
            nb_acc[:] = nb_acc[:] + jnp.reshape(nb_part, (1, 1))

    @pl.when(i == B)
    def _tail():
        q = q_scr[:]
        one_plus = op_scr[:]
        rs = rs_scr[:]
        q2 = q * q
        p_un = q2 * (1.0 / cs_scr[:])
        ps = jnp.maximum(jnp.sum(p_un, axis=1, keepdims=True), 1e-12)
        inv_ps = 1.0 / ps
        lg = jnp.log((p_un * rs) * one_plus)
        a_row = jnp.sum(p_un * lg, axis=1, keepdims=True)
        kl = (jnp.sum(a_row * inv_ps) + jnp.sum(jnp.log(inv_ps))) / ROWS

        gn = jax.lax.dot_general(
            jnp.concatenate([n * -2.0, nn2_col, ones_n], axis=1), n_aug,
            (((1,), (1,)), ((), ())))
        sd = jnp.maximum(gn, 0.0)
        ri = jax.lax.broadcasted_iota(jnp.int32, (NODES, NODES), 0)
        ci = jax.lax.broadcasted_iota(jnp.int32, (NODES, NODES), 1)
        distn = jnp.where(ri == ci, 0.0, jnp.sqrt(sd))
        div = -(jnp.sum(distn) / (NODES * NODES))

        ts = (ts_acc[0, 0] / (B * (T - 1) * DIM)) * TIME_DECAY
        nb = nb_acc[0, 0] / (B * (T - 1))
        tot = kl + 0.5 * div + 0.3 * ts + 0.2 * nb

        kl_ref[:] = jnp.reshape(kl, (1, 1))
        div_ref[:] = jnp.reshape(div, (1, 1))
        ts_ref[:] = jnp.reshape(ts, (1, 1))
        nb_ref[:] = jnp.reshape(nb, (1, 1))
        tot_ref[:] = jnp.reshape(tot, (1, 1))


@functools.partial(jax.jit, static_argnames=("interpret",))
def _run(z, nodes, time_weights, interpret=False):
    nodes_flat = nodes.reshape(NODES, DIM)
    tw = time_weights[0, -T:, :]

    f1 = jax.ShapeDtypeStruct((1, 1), jnp.float32)
    out_shapes = (
        jax.ShapeDtypeStruct((B, T, DIM), jnp.float32),
        jax.ShapeDtypeStruct((ROWS, NODES), jnp.float32),
        jax.ShapeDtypeStruct((B, 1, T), jnp.int32),
        f1, f1, f1, f1, f1,
    )
    last = B - 1
    blk = lambda i: jnp.minimum(i, last)
    f1spec = pl.BlockSpec((1, 1), lambda i: (0, 0))
    som_z, q, bmu, kl, div, ts, nb, tot = pl.pallas_call(
        _som_kernel,
        grid=(B + 1,),
        in_specs=[
            pl.BlockSpec((1, T, DIM), lambda i: (blk(i), 0, 0)),
            pl.BlockSpec((NODES, DIM), lambda i: (0, 0)),
            pl.BlockSpec((T, 1), lambda i: (0, 0)),
        ],
        out_specs=(
            pl.BlockSpec((1, T, DIM), lambda i: (blk(i), 0, 0)),
            pl.BlockSpec((T, NODES), lambda i: (blk(i), 0)),
            pl.BlockSpec((1, 1, T), lambda i: (blk(i), 0, 0)),
            f1spec, f1spec, f1spec, f1spec, f1spec,
        ),
        scratch_shapes=[
            pltpu.VMEM((ROWS, NODES), jnp.float32),
            pltpu.VMEM((ROWS, NODES), jnp.float32),
            pltpu.VMEM((ROWS, 1), jnp.float32),
            pltpu.VMEM((1, NODES), jnp.float32),
            pltpu.VMEM((1, 1), jnp.float32),
            pltpu.VMEM((1, 1), jnp.float32),
        ],
        out_shape=out_shapes,
        interpret=interpret,
    )(z, nodes_flat, tw)

    sc = lambda a: jnp.reshape(a, ())
    return (som_z, sc(tot), sc(kl), sc(div), sc(ts), sc(nb), q,
            bmu.reshape(B, T))


def kernel(z, nodes, time_weights):
    return _run(z, nodes, time_weights)


# speedup vs baseline: 1.1268x; 1.1268x over previous
import functools

import jax
import jax.numpy as jnp
from jax.experimental import pallas as pl
from jax.experimental.pallas import tpu as pltpu

GRID_H = 16
GRID_W = 32
NODES = GRID_H * GRID_W
DIM = 64
B = 4
T = 512
ROWS = B * T
TIME_DECAY = 0.9

_HI = jax.lax.Precision.HIGHEST


def _som_kernel(z_ref, n_ref, tw_ref, som_ref, q_ref, bmu_ref,
                kl_ref, div_ref, ts_ref, nb_ref, tot_ref,
                somv_ref, sem_s):
    z = z_ref[:].reshape(ROWS, DIM)
    n = n_ref[:]
    tw = tw_ref[:]

    tw_full = jnp.concatenate([tw, tw, tw, tw], axis=0)
    wz = z * tw_full
    zn2 = jnp.sum(wz * wz, axis=1, keepdims=True)
    nsq = n * n
    nn2_col = jnp.sum(nsq, axis=1, keepdims=True)

    ones_r = jnp.ones((ROWS, 1), jnp.float32)
    ones_n = jnp.ones((NODES, 1), jnp.float32)
    m_aug = jnp.concatenate([wz * -2.0, zn2, ones_r], axis=1)
    n_aug = jnp.concatenate([n, ones_n, nn2_col], axis=1)
    s = jax.lax.dot_general(
        m_aug, n_aug, (((1,), (1,)), ((), ())), precision=_HI)
    s = jnp.maximum(s, 0.0)
    dist = jnp.sqrt(s)

    one_plus = 1.0 + dist
    q_un = 1.0 / one_plus
    rs = jnp.maximum(jnp.sum(q_un, axis=1, keepdims=True), 1e-12)
    q = q_un * (1.0 / rs)
    q_ref[:] = q

    ids = jax.lax.broadcasted_iota(jnp.int32, (ROWS, NODES), 1)
    mind = jnp.min(dist, axis=1, keepdims=True)
    bmu = jnp.min(jnp.where(dist == mind, ids, NODES), axis=1, keepdims=True)
    for b in range(B):
        bmu_ref[b:b + 1, :] = jnp.transpose(bmu[b * T:(b + 1) * T, :])

    one_hot = (ids == bmu).astype(jnp.float32).astype(jnp.bfloat16)
    gath = jax.lax.dot_general(
        one_hot, n.astype(jnp.bfloat16), (((1,), (0,)), ((), ())),
        preferred_element_type=jnp.float32)
    somv_ref[:] = (z + 0.1 * (gath - z)).reshape(B, T, DIM)
    som_copy = pltpu.make_async_copy(somv_ref, som_ref, sem_s)
    som_copy.start()

    gn = jax.lax.dot_general(
        jnp.concatenate([n * -2.0, nn2_col, ones_n], axis=1), n_aug,
        (((1,), (1,)), ((), ())))
    sd = jnp.maximum(gn, 0.0)
    ri = jax.lax.broadcasted_iota(jnp.int32, (NODES, NODES), 0)
    ci = jax.lax.broadcasted_iota(jnp.int32, (NODES, NODES), 1)
    distn = jnp.where(ri == ci, 0.0, jnp.sqrt(sd))
    div = -(jnp.sum(distn) / (NODES * NODES))

    rid = jax.lax.broadcasted_iota(jnp.int32, (ROWS, 1), 0)
    valid = ((rid & (T - 1)) != (T - 1)).astype(jnp.float32)

    z_next = pltpu.roll(z, shift=ROWS - 1, axis=0)
    dz = z_next - z
    ts = (jnp.sum((dz * dz) * valid) / (B * (T - 1) * DIM)) * TIME_DECAY

    bmu_next = pltpu.roll(bmu, shift=ROWS - 1, axis=0)
    dr = jnp.abs((bmu >> 5) - (bmu_next >> 5))
    dc = jnp.abs((bmu & (GRID_W - 1)) - (bmu_next & (GRID_W - 1)))
    nb = jnp.sum((dr + dc).astype(jnp.float32) * valid) / (B * (T - 1))

    q2 = q * q
    p_un = q2 * (1.0 / jnp.sum(q2, axis=0, keepdims=True))
    ps = jnp.maximum(jnp.sum(p_un, axis=1, keepdims=True), 1e-12)
    inv_ps = 1.0 / ps
    lg = jnp.log((p_un * rs) * one_plus)
    a_row = jnp.sum(p_un * lg, axis=1, keepdims=True)
    kl = (jnp.sum(a_row * inv_ps) + jnp.sum(jnp.log(inv_ps))) / ROWS

    tot = kl + 0.5 * div + 0.3 * ts + 0.2 * nb

    kl_ref[:] = jnp.reshape(kl, (1, 1))
    div_ref[:] = jnp.reshape(div, (1, 1))
    ts_ref[:] = jnp.reshape(ts, (1, 1))
    nb_ref[:] = jnp.reshape(nb, (1, 1))
    tot_ref[:] = jnp.reshape(tot, (1, 1))

    som_copy.wait()


@functools.partial(jax.jit, static_argnames=("interpret",))
def _run(z, nodes, time_weights, interpret=False):
    nodes_flat = nodes.reshape(NODES, DIM)
    tw = time_weights[0, -T:, :]

    f1 = jax.ShapeDtypeStruct((1, 1), jnp.float32)
    out_shapes = (
        jax.ShapeDtypeStruct((B, T, DIM), jnp.float32),
        jax.ShapeDtypeStruct((ROWS, NODES), jnp.float32),
        jax.ShapeDtypeStruct((B, T), jnp.int32),
        f1, f1, f1, f1, f1,
    )
    vspec = pl.BlockSpec(memory_space=pltpu.VMEM)
    aspec = pl.BlockSpec(memory_space=pl.ANY)
    som_z, q, bmu, kl, div, ts, nb, tot = pl.pallas_call(
        _som_kernel,
        out_shape=out_shapes,
        in_specs=[vspec, vspec, vspec],
        out_specs=(aspec, vspec, vspec, vspec, vspec, vspec, vspec, vspec),
        scratch_shapes=[
            pltpu.VMEM((B, T, DIM), jnp.float32),
            pltpu.SemaphoreType.DMA,
        ],
        interpret=interpret,
    )(z, nodes_flat, tw)

    sc = lambda a: jnp.reshape(a, ())
    return (som_z, sc(tot), sc(kl), sc(div), sc(ts), sc(nb), q, bmu)


def kernel(z, nodes, time_weights):
    return _run(z, nodes, time_weights)


# speedup vs baseline: 1.2433x; 1.1035x over previous
import functools

import jax
import jax.numpy as jnp
from jax.experimental import pallas as pl
from jax.experimental.pallas import tpu as pltpu

GRID_H = 16
GRID_W = 32
NODES = GRID_H * GRID_W
DIM = 64
B = 4
T = 512
ROWS = B * T
TIME_DECAY = 0.9

_HI = jax.lax.Precision.HIGHEST


def _som_kernel(z_ref, n_ref, tw_ref, som_ref, q_ref, bmu_ref,
                kl_ref, div_ref, ts_ref, nb_ref, tot_ref):
    z = z_ref[:].reshape(ROWS, DIM)
    n = n_ref[:]
    tw = tw_ref[:]

    tw_full = jnp.concatenate([tw, tw, tw, tw], axis=0)
    wz = z * tw_full
    zn2 = jnp.sum(wz * wz, axis=1, keepdims=True)
    nsq = n * n
    nn2_col = jnp.sum(nsq, axis=1, keepdims=True)

    ones_r = jnp.ones((ROWS, 1), jnp.float32)
    ones_n = jnp.ones((NODES, 1), jnp.float32)
    m_aug = jnp.concatenate([wz * -2.0, zn2, ones_r], axis=1)
    n_aug = jnp.concatenate([n, ones_n, nn2_col], axis=1)
    s = jax.lax.dot_general(
        m_aug, n_aug, (((1,), (1,)), ((), ())), precision=_HI)
    s = jnp.maximum(s, 0.0)
    dist = jnp.sqrt(s)

    one_plus = 1.0 + dist
    q_un = 1.0 / one_plus
    rs = jnp.maximum(jnp.sum(q_un, axis=1, keepdims=True), 1e-12)
    q = q_un * (1.0 / rs)
    q_ref[:] = q

    q2 = q * q
    p_un = q2 * (1.0 / jnp.sum(q2, axis=0, keepdims=True))
    ps = jnp.maximum(jnp.sum(p_un, axis=1, keepdims=True), 1e-12)
    inv_ps = 1.0 / ps
    lg = jnp.log((p_un * rs) * one_plus)
    a_row = jnp.sum(p_un * lg, axis=1, keepdims=True)
    kl = (jnp.sum(a_row * inv_ps) + jnp.sum(jnp.log(inv_ps))) / ROWS

    ids = jax.lax.broadcasted_iota(jnp.int32, (ROWS, NODES), 1)
    mind = jnp.min(dist, axis=1, keepdims=True)
    bmu = jnp.min(jnp.where(dist == mind, ids, NODES), axis=1, keepdims=True)
    for b in range(B):
        bmu_ref[b:b + 1, :] = jnp.transpose(bmu[b * T:(b + 1) * T, :])

    one_hot = (ids == bmu).astype(jnp.float32).astype(jnp.bfloat16)
    gath = jax.lax.dot_general(
        one_hot, n.astype(jnp.bfloat16), (((1,), (0,)), ((), ())),
        preferred_element_type=jnp.float32)
    som_ref[:] = (z + 0.1 * (gath - z)).reshape(B, T, DIM)

    gn = jax.lax.dot_general(
        jnp.concatenate([n * -2.0, nn2_col, ones_n], axis=1), n_aug,
        (((1,), (1,)), ((), ())))
    sd = jnp.maximum(gn, 0.0)
    ri = jax.lax.broadcasted_iota(jnp.int32, (NODES, NODES), 0)
    ci = jax.lax.broadcasted_iota(jnp.int32, (NODES, NODES), 1)
    distn = jnp.where(ri == ci, 0.0, jnp.sqrt(sd))
    div = -(jnp.sum(distn) / (NODES * NODES))

    rid = jax.lax.broadcasted_iota(jnp.int32, (ROWS, 1), 0)
    valid = ((rid & (T - 1)) != (T - 1)).astype(jnp.float32)

    z_next = pltpu.roll(z, shift=ROWS - 1, axis=0)
    dz = z_next - z
    ts = (jnp.sum((dz * dz) * valid) / (B * (T - 1) * DIM)) * TIME_DECAY

    bmu_next = pltpu.roll(bmu, shift=ROWS - 1, axis=0)
    dr = jnp.abs((bmu >> 5) - (bmu_next >> 5))
    dc = jnp.abs((bmu & (GRID_W - 1)) - (bmu_next & (GRID_W - 1)))
    nb = jnp.sum((dr + dc).astype(jnp.float32) * valid) / (B * (T - 1))

    tot = kl + 0.5 * div + 0.3 * ts + 0.2 * nb

    kl_ref[:] = jnp.reshape(kl, (1, 1))
    div_ref[:] = jnp.reshape(div, (1, 1))
    ts_ref[:] = jnp.reshape(ts, (1, 1))
    nb_ref[:] = jnp.reshape(nb, (1, 1))
    tot_ref[:] = jnp.reshape(tot, (1, 1))


@functools.partial(jax.jit, static_argnames=("interpret",))
def _run(z, nodes, time_weights, interpret=False):
    nodes_flat = nodes.reshape(NODES, DIM)
    tw = time_weights[0, -T:, :]

    f1 = jax.ShapeDtypeStruct((1, 1), jnp.float32)
    out_shapes = (
        jax.ShapeDtypeStruct((B, T, DIM), jnp.float32),
        jax.ShapeDtypeStruct((ROWS, NODES), jnp.float32),
        jax.ShapeDtypeStruct((B, T), jnp.int32),
        f1, f1, f1, f1, f1,
    )
    som_z, q, bmu, kl, div, ts, nb, tot = pl.pallas_call(
        _som_kernel,
        out_shape=out_shapes,
        interpret=interpret,
    )(z, nodes_flat, tw)

    sc = lambda a: jnp.reshape(a, ())
    return (som_z, sc(tot), sc(kl), sc(div), sc(ts), sc(nb), q, bmu)


def kernel(z, nodes, time_weights):
    return _run(z, nodes, time_weights)
